# Initial kernel scaffold; baseline (speedup 1.0000x reference)
import jax
import jax.numpy as jnp
from jax.experimental import pallas as pl


def kernel(x, table):
    raise NotImplementedError("write your pallas kernel here")


# speedup vs baseline: 1.4426x; 1.4426x over previous
import functools

import jax
import jax.numpy as jnp
from jax import lax
from jax.experimental import pallas as pl
from jax.experimental.pallas import tpu as pltpu
from jax.experimental.pallas import tpu_sc as plsc

_LANES = 16
_CHUNK = 512


def _sc_geometry():
    try:
        info = plsc.get_sparse_core_info()
        return info.num_cores, info.num_subcores
    except Exception:
        return 2, 16


def _lane_sum(x):
    lanes = lax.iota(jnp.int32, _LANES)
    dnums = lax.GatherDimensionNumbers(
        offset_dims=(), collapsed_slice_dims=(0,), start_index_map=(0,))
    for sh in (8, 4, 2, 1):
        perm = jnp.bitwise_xor(lanes, sh)
        x = x + lax.gather(
            x, perm[:, None], dnums, (1,),
            mode=lax.GatherScatterMode.PROMISE_IN_BOUNDS)
    return x


def _rsqrt_vec(s):
    i = lax.bitcast_convert_type(s, jnp.int32)
    i = jnp.int32(0x5F3759DF) - lax.shift_right_logical(i, 1)
    y = lax.bitcast_convert_type(i, jnp.float32)
    for _ in range(3):
        y = y * (1.5 - 0.5 * s * y * y)
    return y


@functools.lru_cache(maxsize=None)
def _make_sc_lookup(n_idx, dim, chunk):
    nc, ns = _sc_geometry()
    nw = nc * ns
    n_per_w = n_idx // nw
    n_chunks = n_per_w // chunk
    n_seg = dim // _LANES
    mesh = plsc.VectorSubcoreMesh(core_axis_name="c", subcore_axis_name="s")

    @functools.partial(
        pl.kernel,
        mesh=mesh,
        compiler_params=pltpu.CompilerParams(use_tc_tiling_on_sc=False),
        out_type=jax.ShapeDtypeStruct((n_idx, dim), jnp.float32),
        scratch_types=[
            pltpu.VMEM((chunk,), jnp.int32),
            pltpu.VMEM((chunk, dim), jnp.float32),
            pltpu.SemaphoreType.DMA,
        ],
    )
    def lookup(idx_hbm, table_hbm, out_hbm, idx_v, rows_v, sem):
        wid = lax.axis_index("s") * nc + lax.axis_index("c")
        w_base = wid * n_per_w

        def chunk_body(c, carry):
            base = w_base + c * chunk
            pltpu.sync_copy(idx_hbm.at[pl.ds(base, chunk)], idx_v)
            pltpu.async_copy(table_hbm.at[idx_v], rows_v, sem).wait()

            def row_body(r, rcarry):
                vs = [rows_v[r, pl.ds(j * _LANES, _LANES)] for j in range(n_seg)]
                ss = vs[0] * vs[0]
                for v in vs[1:]:
                    ss = ss + v * v
                sv = _lane_sum(ss)
                scale = jnp.minimum(_rsqrt_vec(sv), 1.0)
                for j, v in enumerate(vs):
                    rows_v[r, pl.ds(j * _LANES, _LANES)] = v * scale
                return rcarry

            lax.fori_loop(0, chunk, row_body, 0, unroll=2)
            pltpu.sync_copy(rows_v, out_hbm.at[pl.ds(base, chunk)])
            return carry

        lax.fori_loop(0, n_chunks, chunk_body, 0)

    return lookup


def kernel(x, table):
    b, h = x.shape
    _, d = table.shape
    n = b * h
    nc, ns = _sc_geometry()
    grain = nc * ns * _CHUNK
    n_pad = -(-n // grain) * grain
    flat = x.reshape(n)
    if n_pad != n:
        flat = jnp.concatenate([flat, jnp.zeros(n_pad - n, jnp.int32)])
    out = _make_sc_lookup(n_pad, d, _CHUNK)(flat, table)
    if n_pad != n:
        out = out[:n]
    return out.reshape(b, h, d)


# speedup vs baseline: 1.6199x; 1.1229x over previous
import functools

import jax
import jax.numpy as jnp
from jax import lax
from jax.experimental import pallas as pl
from jax.experimental.pallas import tpu as pltpu
from jax.experimental.pallas import tpu_sc as plsc

_LANES = 16
_CHUNK = 256
_NBUF = 4


def _sc_geometry():
    try:
        info = plsc.get_sparse_core_info()
        return info.num_cores, info.num_subcores
    except Exception:
        return 2, 16


def _lane_sum(x):
    lanes = lax.iota(jnp.int32, _LANES)
    dnums = lax.GatherDimensionNumbers(
        offset_dims=(), collapsed_slice_dims=(0,), start_index_map=(0,))
    for sh in (8, 4, 2, 1):
        perm = jnp.bitwise_xor(lanes, sh)
        x = x + lax.gather(
            x, perm[:, None], dnums, (1,),
            mode=lax.GatherScatterMode.PROMISE_IN_BOUNDS)
    return x


def _rsqrt_vec(s):
    i = lax.bitcast_convert_type(s, jnp.int32)
    i = jnp.int32(0x5F3759DF) - lax.shift_right_logical(i, 1)
    y = lax.bitcast_convert_type(i, jnp.float32)
    for _ in range(3):
        y = y * (1.5 - 0.5 * s * y * y)
    return y


@functools.lru_cache(maxsize=None)
def _make_sc_lookup(n_idx, dim, chunk, nbuf):
    nc, ns = _sc_geometry()
    nw = nc * ns
    n_per_w = n_idx // nw
    n_chunks = n_per_w // chunk
    n_groups = n_chunks // nbuf
    n_seg = dim // _LANES
    mesh = plsc.VectorSubcoreMesh(core_axis_name="c", subcore_axis_name="s")

    @functools.partial(
        pl.kernel,
        mesh=mesh,
        compiler_params=pltpu.CompilerParams(use_tc_tiling_on_sc=False),
        out_type=jax.ShapeDtypeStruct((n_idx, dim), jnp.float32),
        scratch_types=[
            pltpu.VMEM((nbuf, chunk), jnp.int32),
            pltpu.VMEM((nbuf, chunk, dim), jnp.float32),
        ] + [pltpu.SemaphoreType.DMA] * (3 * nbuf),
    )
    def lookup(idx_hbm, table_hbm, out_hbm, idx_v, rows_v, *sems):
        sem_i = sems[0:nbuf]
        sem_g = sems[nbuf:2 * nbuf]
        sem_w = sems[2 * nbuf:3 * nbuf]
        wid = lax.axis_index("s") * nc + lax.axis_index("c")
        w_base = wid * n_per_w

        def idx_dma(c, b):
            return pltpu.make_async_copy(
                idx_hbm.at[pl.ds(w_base + c * chunk, chunk)],
                idx_v.at[b], sem_i[b])

        def gather_dma(b):
            return pltpu.make_async_copy(
                table_hbm.at[idx_v.at[b]], rows_v.at[b], sem_g[b])

        def wb_dma(c, b):
            return pltpu.make_async_copy(
                rows_v.at[b], out_hbm.at[pl.ds(w_base + c * chunk, chunk)],
                sem_w[b])

        for c in range(min(nbuf, n_chunks)):
            idx_dma(c, c % nbuf).start()
        for c in range(min(2, n_chunks)):
            b = c % nbuf
            idx_dma(c, b).wait()
            gather_dma(b).start()

        def compute_rows(b):
            def row_body(r, rcarry):
                vs = [rows_v[b, r, pl.ds(j * _LANES, _LANES)]
                      for j in range(n_seg)]
                ss = vs[0] * vs[0]
                for v in vs[1:]:
                    ss = ss + v * v
                scale = jnp.minimum(_rsqrt_vec(_lane_sum(ss)), 1.0)
                for j, v in enumerate(vs):
                    rows_v[b, r, pl.ds(j * _LANES, _LANES)] = v * scale
                return rcarry

            lax.fori_loop(0, chunk, row_body, 0, unroll=2)

        def group_body(g, carry):
            for b in range(nbuf):
                c = g * nbuf + b
                b2 = (b + 2) % nbuf
                gather_dma(b).wait()
                @pl.when(c + 2 < n_chunks)
                def _():
                    @pl.when(c >= 2)
                    def _():
                        wb_dma(c - 2, b2).wait()
                    idx_dma(c + 2, b2).wait()
                    gather_dma(b2).start()

                @pl.when(c + nbuf < n_chunks)
                def _():
                    idx_dma(c + nbuf, b).start()
                compute_rows(b)
                wb_dma(c, b).start()
            return carry

        lax.fori_loop(0, n_groups, group_body, 0)
        for c in (n_chunks - 2, n_chunks - 1):
            if c >= 0:
                wb_dma(c, c % nbuf).wait()

    return lookup


def kernel(x, table):
    b, h = x.shape
    _, d = table.shape
    n = b * h
    nc, ns = _sc_geometry()
    grain = nc * ns * _CHUNK * _NBUF
    n_pad = -(-n // grain) * grain
    flat = x.reshape(n)
    if n_pad != n:
        flat = jnp.concatenate([flat, jnp.zeros(n_pad - n, jnp.int32)])
    out = _make_sc_lookup(n_pad, d, _CHUNK, _NBUF)(flat, table)
    if n_pad != n:
        out = out[:n]
    return out.reshape(b, h, d)


# speedup vs baseline: 2.0294x; 1.2528x over previous
import functools

import jax
import jax.numpy as jnp
from jax import lax
from jax.experimental import pallas as pl
from jax.experimental.pallas import tpu as pltpu
from jax.experimental.pallas import tpu_sc as plsc

_LANES = 16
_CHUNK = 256


def _sc_geometry():
    try:
        info = plsc.get_sparse_core_info()
        return info.num_cores, info.num_subcores
    except Exception:
        return 2, 16


def _rsqrt_vec(s):
    i = lax.bitcast_convert_type(s, jnp.int32)
    i = jnp.int32(0x5F3759DF) - lax.shift_right_logical(i, 1)
    y = lax.bitcast_convert_type(i, jnp.float32)
    for _ in range(3):
        y = y * (1.5 - 0.5 * s * y * y)
    return y


@functools.lru_cache(maxsize=None)
def _make_sc_lookup(hist, batch, dim, chunk):
    nc, ns = _sc_geometry()
    nw = nc * ns
    strip = batch // nw
    cps = strip // chunk
    n_chunks = hist * cps
    dmask = dim - 1
    mesh = plsc.VectorSubcoreMesh(core_axis_name="c", subcore_axis_name="s")

    @functools.partial(
        pl.kernel,
        mesh=mesh,
        compiler_params=pltpu.CompilerParams(
            use_tc_tiling_on_sc=False, needs_layout_passes=False),
        out_type=jax.ShapeDtypeStruct((hist, dim, batch), jnp.float32),
        scratch_types=[
            pltpu.VMEM((2, chunk), jnp.int32),
            pltpu.VMEM((2, chunk, dim), jnp.float32),
            pltpu.VMEM((2, dim, chunk), jnp.float32),
        ] + [pltpu.SemaphoreType.DMA] * 6,
    )
    def lookup(xt_hbm, table_hbm, out_hbm, idx_v, rows_v, cols_v, *sems):
        sem_i, sem_g, sem_w = sems[0:2], sems[2:4], sems[4:6]
        wid = lax.axis_index("s") * nc + lax.axis_index("c")
        w_base = wid * strip
        lanes = lax.iota(jnp.int32, _LANES)

        def chunk_hb(k):
            return k // cps, w_base + (k % cps) * chunk

        def idx_dma(k, p):
            h, b0 = chunk_hb(k)
            return pltpu.make_async_copy(
                xt_hbm.at[h, pl.ds(b0, chunk)], idx_v.at[p], sem_i[p])

        def gather_dma(p):
            return pltpu.make_async_copy(
                table_hbm.at[idx_v.at[p]], rows_v.at[p], sem_g[p])

        def wb_dma(k, p):
            h, b0 = chunk_hb(k)
            return pltpu.make_async_copy(
                cols_v.at[p], out_hbm.at[h, :, pl.ds(b0, chunk)], sem_w[p])

        def compute(p):
            rows = rows_v.at[p]
            cols = cols_v.at[p]

            def group_body(g, gcarry):
                r16 = g * _LANES + lanes

                def pass_a(t, carry):
                    ss0, ss1, col = carry
                    v0 = plsc.load_gather(rows, [r16, col])
                    v1 = plsc.load_gather(rows, [r16, (col + dim // 2) & dmask])
                    return ss0 + v0 * v0, ss1 + v1 * v1, (col + 1) & dmask

                ss0, ss1, _ = lax.fori_loop(
                    0, dim // 2, pass_a,
                    (jnp.zeros(_LANES, jnp.float32),
                     jnp.zeros(_LANES, jnp.float32), lanes),
                    unroll=4)
                scale = jnp.minimum(_rsqrt_vec(ss0 + ss1), 1.0)

                def pass_b(t, col):
                    c1 = (col + dim // 2) & dmask
                    v0 = plsc.load_gather(rows, [r16, col])
                    v1 = plsc.load_gather(rows, [r16, c1])
                    plsc.store_scatter(cols, [col, r16], v0 * scale)
                    plsc.store_scatter(cols, [c1, r16], v1 * scale)
                    return (col + 1) & dmask

                lax.fori_loop(0, dim // 2, pass_b, lanes, unroll=4)
                return gcarry

            lax.fori_loop(0, chunk // _LANES, group_body, 0)

        idx_dma(0, 0).start()
        idx_dma(1, 1).start()
        idx_dma(0, 0).wait()
        gather_dma(0).start()

        def pair_body(q, carry):
            for p in (0, 1):
                k = 2 * q + p
                gather_dma(p).wait()

                @pl.when(k + 1 < n_chunks)
                def _():
                    idx_dma(k + 1, 1 - p).wait()
                    gather_dma(1 - p).start()

                @pl.when(k + 2 < n_chunks)
                def _():
                    idx_dma(k + 2, p).start()

                @pl.when(k >= 2)
                def _():
                    wb_dma(k - 2, p).wait()
                compute(p)
                wb_dma(k, p).start()
            return carry

        lax.fori_loop(0, n_chunks // 2, pair_body, 0)
        wb_dma(n_chunks - 2, 0).wait()
        wb_dma(n_chunks - 1, 1).wait()

    return lookup


def kernel(x, table):
    b, h = x.shape
    _, d = table.shape
    nc, ns = _sc_geometry()
    grain = nc * ns * _CHUNK * 2
    b_pad = -(-b // grain) * grain
    xt = jnp.swapaxes(x, 0, 1)
    if b_pad != b:
        xt = jnp.pad(xt, ((0, 0), (0, b_pad - b)))
    out_t = _make_sc_lookup(h, b_pad, d, _CHUNK)(xt, table)
    if b_pad != b:
        out_t = out_t[:, :, :b]
    return jnp.transpose(out_t, (2, 0, 1))


# speedup vs baseline: 2.5978x; 1.2801x over previous
import functools

import jax
import jax.numpy as jnp
from jax import lax
from jax.experimental import pallas as pl
from jax.experimental.pallas import tpu as pltpu
from jax.experimental.pallas import tpu_sc as plsc

_LANES = 16
_CHUNK = 256


def _sc_geometry():
    try:
        info = plsc.get_sparse_core_info()
        return info.num_cores, info.num_subcores
    except Exception:
        return 2, 16


def _rsqrt_vec(s):
    i = lax.bitcast_convert_type(s, jnp.int32)
    i = jnp.int32(0x5F3759DF) - lax.shift_right_logical(i, 1)
    y = lax.bitcast_convert_type(i, jnp.float32)
    for _ in range(3):
        y = y * (1.5 - 0.5 * s * y * y)
    return y


@functools.lru_cache(maxsize=None)
def _make_table_transpose(vocab, dim):
    nc, ns = _sc_geometry()
    nw = nc * ns
    n_full = vocab // 128
    w_tail = vocab % 128
    per_w = -(-n_full // nw)
    n_pair = -(-(per_w + 2) // 2)
    mesh = plsc.VectorSubcoreMesh(core_axis_name="c", subcore_axis_name="s")

    @functools.partial(
        pl.kernel,
        mesh=mesh,
        compiler_params=pltpu.CompilerParams(
            use_tc_tiling_on_sc=True, needs_layout_passes=False),
        out_type=jax.ShapeDtypeStruct((vocab // 2, 128), jnp.float32),
        scratch_types=[
            pltpu.VMEM((2, dim, 128), jnp.float32),
            pltpu.VMEM((2, dim, 128), jnp.float32),
        ] + [pltpu.SemaphoreType.DMA] * 4,
    )
    def transpose_k(tt_hbm, tailp_hbm, t2_hbm, stage_v, blk_v, *sems):
        sem_i, sem_w = sems[0:2], sems[2:4]
        wid = lax.axis_index("s") * nc + lax.axis_index("c")
        lanes = lax.iota(jnp.int32, _LANES)

        def blk_of(j):
            return wid + j * nw

        def in_dma(j, p):
            return pltpu.make_async_copy(
                tt_hbm.at[:, pl.ds(blk_of(j) * 128, 128)], stage_v.at[p],
                sem_i[p])

        def wb_dma(j, p):
            return pltpu.make_async_copy(
                blk_v.at[p], t2_hbm.at[pl.ds(blk_of(j) * (128 // 2), dim), :],
                sem_w[p])

        def transpose_block(src, dst, ncols):
            for c0 in range(0, ncols, _LANES):
                cvec = c0 + lanes
                vr = lax.shift_right_logical(cvec, 1)
                jbase = lax.shift_left(
                    jnp.bitwise_and(cvec, 1), jnp.int32(6))

                def tbody(t, dvec):
                    v = plsc.load_gather(src, [dvec, cvec])
                    plsc.store_scatter(
                        dst, [vr, jnp.bitwise_or(jbase, dvec)], v)
                    return jnp.bitwise_and(dvec + 1, dim - 1)

                lax.fori_loop(0, dim, tbody, lanes, unroll=4)

        @pl.when(blk_of(0) < n_full)
        def _():
            in_dma(0, 0).start()

        def pair_body(q, carry):
            for p in (0, 1):
                j = 2 * q + p
                valid = blk_of(j) < n_full

                @pl.when(valid)
                def _():
                    in_dma(j, p).wait()

                @pl.when(blk_of(j + 1) < n_full)
                def _():
                    in_dma(j + 1, 1 - p).start()

                @pl.when(jnp.logical_and(j >= 2, blk_of(j - 2) < n_full))
                def _():
                    wb_dma(j - 2, p).wait()

                @pl.when(valid)
                def _():
                    transpose_block(stage_v.at[p], blk_v.at[p], 128)
                    wb_dma(j, p).start()
            return carry

        lax.fori_loop(0, n_pair, pair_body, 0)

        if w_tail:
            @pl.when(wid == nw - 1)
            def _():
                pltpu.sync_copy(tailp_hbm, stage_v.at[0])
                transpose_block(stage_v.at[0], blk_v.at[0], w_tail)
                pltpu.sync_copy(
                    blk_v.at[0, pl.ds(0, w_tail // 2), :],
                    t2_hbm.at[pl.ds(n_full * (128 // 2), w_tail // 2), :])

    return transpose_k


@functools.lru_cache(maxsize=None)
def _make_sc_lookup(hist, batch, dim, chunk):
    nc, ns = _sc_geometry()
    nw = nc * ns
    strip = batch // nw
    cps = strip // chunk
    n_chunks = hist * cps
    dmask = dim - 1
    ntr = dim // 8
    ntc = chunk // 128
    mesh = plsc.VectorSubcoreMesh(core_axis_name="c", subcore_axis_name="s")

    @functools.partial(
        pl.kernel,
        mesh=mesh,
        compiler_params=pltpu.CompilerParams(
            use_tc_tiling_on_sc=False, needs_layout_passes=False),
        out_type=jax.ShapeDtypeStruct(
            (hist, ntr, batch // 128, 8, 128), jnp.float32),
        scratch_types=[
            pltpu.VMEM((2, chunk), jnp.int32),
            pltpu.VMEM((2, chunk, dim), jnp.float32),
            pltpu.VMEM((2, dim, chunk), jnp.float32),
        ] + [pltpu.SemaphoreType.DMA] * 6,
    )
    def lookup(xt_hbm, table_hbm, out_hbm, idx_v, rows_v, cols_v, *sems):
        sem_i, sem_g, sem_w = sems[0:2], sems[2:4], sems[4:6]
        wid = lax.axis_index("s") * nc + lax.axis_index("c")
        w_base = wid * strip
        lanes = lax.iota(jnp.int32, _LANES)

        def chunk_hb(k):
            return k // cps, w_base + (k % cps) * chunk

        def idx_dma(k, p):
            h, b0 = chunk_hb(k)
            return pltpu.make_async_copy(
                xt_hbm.at[h, pl.ds(b0, chunk)], idx_v.at[p], sem_i[p])

        def gather_dma(p):
            return pltpu.make_async_copy(
                table_hbm.at[idx_v.at[p]], rows_v.at[p], sem_g[p])

        def wb_dmas(k, p):
            h, b0 = chunk_hb(k)
            tc0 = b0 // 128
            return [
                pltpu.make_async_copy(
                    cols_v.at[p, pl.ds(tr * 8, 8), pl.ds(tc * 128, 128)],
                    out_hbm.at[h, tr, tc0 + tc, :, :], sem_w[p])
                for tr in range(ntr) for tc in range(ntc)]

        def compute(p):
            rows = rows_v.at[p]
            cols = cols_v.at[p]

            def group_body(g, gcarry):
                r16 = g * _LANES + lanes

                def pass_a(t, carry):
                    ss0, ss1, col = carry
                    v0 = plsc.load_gather(rows, [r16, col])
                    v1 = plsc.load_gather(rows, [r16, (col + dim // 2) & dmask])
                    return ss0 + v0 * v0, ss1 + v1 * v1, (col + 1) & dmask

                ss0, ss1, _ = lax.fori_loop(
                    0, dim // 2, pass_a,
                    (jnp.zeros(_LANES, jnp.float32),
                     jnp.zeros(_LANES, jnp.float32), lanes),
                    unroll=4)
                scale = jnp.minimum(_rsqrt_vec(ss0 + ss1), 1.0)

                def pass_b(t, col):
                    c1 = (col + dim // 2) & dmask
                    v0 = plsc.load_gather(rows, [r16, col])
                    v1 = plsc.load_gather(rows, [r16, c1])
                    plsc.store_scatter(cols, [col, r16], v0 * scale)
                    plsc.store_scatter(cols, [c1, r16], v1 * scale)
                    return (col + 1) & dmask

                lax.fori_loop(0, dim // 2, pass_b, lanes, unroll=4)
                return gcarry

            lax.fori_loop(0, chunk // _LANES, group_body, 0)

        idx_dma(0, 0).start()
        idx_dma(1, 1).start()
        idx_dma(0, 0).wait()
        gather_dma(0).start()

        def pair_body(q, carry):
            for p in (0, 1):
                k = 2 * q + p
                gather_dma(p).wait()

                @pl.when(k + 1 < n_chunks)
                def _():
                    idx_dma(k + 1, 1 - p).wait()
                    gather_dma(1 - p).start()

                @pl.when(k + 2 < n_chunks)
                def _():
                    idx_dma(k + 2, p).start()

                @pl.when(k >= 2)
                def _():
                    for dmac in wb_dmas(k - 2, p):
                        dmac.wait()
                compute(p)
                for dmac in wb_dmas(k, p):
                    dmac.start()
            return carry

        lax.fori_loop(0, n_chunks // 2, pair_body, 0)
        for dmac in wb_dmas(n_chunks - 2, 0) + wb_dmas(n_chunks - 1, 1):
            dmac.wait()

    return lookup


def kernel(x, table):
    b, h = x.shape
    v, d = table.shape
    nc, ns = _sc_geometry()
    grain = nc * ns * _CHUNK * 2
    b_pad = -(-b // grain) * grain
    xt = jnp.swapaxes(x, 0, 1)
    if b_pad != b:
        xt = jnp.pad(xt, ((0, 0), (0, b_pad - b)))
    tt = jnp.swapaxes(table, 0, 1)
    tail_p = jnp.pad(tt[:, (v // 128) * 128:],
                     ((0, 0), (0, 128 - v % 128 if v % 128 else 128)))
    t2 = _make_table_transpose(v, d)(tt, tail_p)
    trm = jnp.reshape(t2, (v, d))
    o5 = _make_sc_lookup(h, b_pad, d, _CHUNK)(xt, trm)
    out_t = jnp.reshape(jnp.transpose(o5, (0, 1, 3, 2, 4)), (h, d, b_pad))
    if b_pad != b:
        out_t = out_t[:, :, :b]
    return jnp.transpose(out_t, (2, 0, 1))
